# Initial kernel scaffold; baseline (speedup 1.0000x reference)
import jax
import jax.numpy as jnp
from jax import lax
from jax.experimental import pallas as pl
from jax.experimental.pallas import tpu as pltpu
from jax.experimental.pallas import tpu_sc as plsc

NUM_USERS = 25000
NUM_ITEMS = 25000
N = NUM_USERS + NUM_ITEMS
E = 800000
D = 64
BATCH = 1024

NC = 2
NS = 16
L = 16

HALF = N // NC
K = 80
EPT = E // NS
NCHUNK = EPT // K
ROWS_A = 1563
ROWS_B = HALF - 15 * ROWS_A

_MESH = plsc.VectorSubcoreMesh(
    core_axis_name="c", subcore_axis_name="s", num_cores=NC, num_subcores=NS
)


def _prop_body(emb_hbm, src_hbm, dst_hbm, w_hbm, zeros_hbm, out_hbm,
               acc, srcbuf, dstbuf, dlbuf, wbuf, rows, sem):
    cid = lax.axis_index("c")
    sid = lax.axis_index("s")
    lo = cid * HALF
    hi = lo + HALF

    @pl.when(sid < NS - 1)
    def _():
        pltpu.sync_copy(zeros_hbm.at[pl.ds(0, ROWS_A)],
                        acc.at[pl.ds(sid * ROWS_A, ROWS_A)])

    @pl.when(sid == NS - 1)
    def _():
        pltpu.sync_copy(zeros_hbm.at[pl.ds(0, ROWS_B)],
                        acc.at[pl.ds(sid * ROWS_A, ROWS_B)])

    plsc.subcore_barrier()

    def chunk_body(i, carry):
        base = sid * EPT + i * K
        pltpu.sync_copy(src_hbm.at[pl.ds(base, K)], srcbuf)
        pltpu.sync_copy(dst_hbm.at[pl.ds(base, K)], dstbuf)
        pltpu.sync_copy(w_hbm.at[pl.ds(base, K)], wbuf)
        pltpu.async_copy(emb_hbm.at[srcbuf], rows, sem).wait()

        for g in range(K // L):
            sl = pl.ds(g * L, L)
            d = dstbuf[sl]
            m = (d >= lo) & (d < hi)
            wbuf[sl] = jnp.where(m, wbuf[sl], 0.0)
            dlbuf[sl] = jnp.where(m, d - lo, 0)

        def row_body(r, c2):
            sw = plsc.load_gather(wbuf, [jnp.full((L,), r, jnp.int32)])
            for c in range(D // L):
                cs = pl.ds(c * L, L)
                rows[r, cs] = rows[r, cs] * sw
            return c2

        lax.fori_loop(0, K, row_body, 0, unroll=4)
        pltpu.sync_copy(rows, acc.at[dlbuf], add=True)
        return carry

    lax.fori_loop(0, NCHUNK, chunk_body, 0)
    plsc.subcore_barrier()

    @pl.when(sid < NS - 1)
    def _():
        pltpu.sync_copy(acc.at[pl.ds(sid * ROWS_A, ROWS_A)],
                        out_hbm.at[pl.ds(lo + sid * ROWS_A, ROWS_A)])

    @pl.when(sid == NS - 1)
    def _():
        pltpu.sync_copy(acc.at[pl.ds(sid * ROWS_A, ROWS_B)],
                        out_hbm.at[pl.ds(lo + sid * ROWS_A, ROWS_B)])


_prop = pl.kernel(
    _prop_body,
    out_type=jax.ShapeDtypeStruct((N, D), jnp.float32),
    mesh=_MESH,
    scratch_types=[
        pltpu.VMEM_SHARED((HALF, D), jnp.float32),
        pltpu.VMEM((K,), jnp.int32),
        pltpu.VMEM((K,), jnp.int32),
        pltpu.VMEM((K,), jnp.int32),
        pltpu.VMEM((K,), jnp.float32),
        pltpu.VMEM((K, D), jnp.float32),
        pltpu.SemaphoreType.DMA,
    ],
)

UPW = BATCH // (NC * NS)


def _users_body(e0u_hbm, e1_hbm, e2_hbm, e3_hbm, users_hbm, out_hbm,
                idxbuf, b0, b1, b2, b3, sem):
    wid = lax.axis_index("s") * NC + lax.axis_index("c")
    base = wid * UPW
    pltpu.sync_copy(users_hbm.at[pl.ds(base, UPW)], idxbuf)
    pltpu.async_copy(e0u_hbm.at[idxbuf], b0, sem).wait()
    pltpu.async_copy(e1_hbm.at[idxbuf], b1, sem).wait()
    pltpu.async_copy(e2_hbm.at[idxbuf], b2, sem).wait()
    pltpu.async_copy(e3_hbm.at[idxbuf], b3, sem).wait()

    def row_body(r, c2):
        for c in range(D // L):
            cs = pl.ds(c * L, L)
            b0[r, cs] = (b0[r, cs] + b1[r, cs] + b2[r, cs] + b3[r, cs]) * 0.25
        return c2

    lax.fori_loop(0, UPW, row_body, 0, unroll=4)
    pltpu.sync_copy(b0, out_hbm.at[pl.ds(base, UPW)])


_users_mean = pl.kernel(
    _users_body,
    out_type=jax.ShapeDtypeStruct((BATCH, D), jnp.float32),
    mesh=_MESH,
    scratch_types=[
        pltpu.VMEM((UPW,), jnp.int32),
        pltpu.VMEM((UPW, D), jnp.float32),
        pltpu.VMEM((UPW, D), jnp.float32),
        pltpu.VMEM((UPW, D), jnp.float32),
        pltpu.VMEM((UPW, D), jnp.float32),
        pltpu.SemaphoreType.DMA,
    ],
)

IB = 3125


def _rating_body(um_ref, i0_ref, i1_ref, i2_ref, i3_ref, out_ref):
    items = (i0_ref[...] + i1_ref[...] + i2_ref[...] + i3_ref[...]) * 0.25
    out_ref[...] = lax.dot_general(
        um_ref[...], items, (((1,), (1,)), ((), ())),
        preferred_element_type=jnp.float32)


def _rating(users_mean, item_emb, e1, e2, e3):
    nblk = NUM_ITEMS // IB
    return pl.pallas_call(
        _rating_body,
        grid=(nblk,),
        in_specs=[
            pl.BlockSpec((BATCH, D), lambda i: (0, 0)),
            pl.BlockSpec((IB, D), lambda i: (i, 0)),
            pl.BlockSpec((IB, D), lambda i: (NUM_USERS // IB + i, 0)),
            pl.BlockSpec((IB, D), lambda i: (NUM_USERS // IB + i, 0)),
            pl.BlockSpec((IB, D), lambda i: (NUM_USERS // IB + i, 0)),
        ],
        out_specs=pl.BlockSpec((BATCH, IB), lambda i: (0, i)),
        out_shape=jax.ShapeDtypeStruct((BATCH, NUM_ITEMS), jnp.float32),
    )(users_mean, item_emb, e1, e2, e3)


@jax.jit
def kernel(user_emb, item_emb, edge_weight, edge_index, users):
    e0 = jnp.concatenate([user_emb, item_emb], axis=0)
    dst = edge_index[0]
    src = edge_index[1]
    zeros = jnp.zeros((ROWS_A, D), jnp.float32)
    e1 = _prop(e0, src, dst, edge_weight, zeros)
    e2 = _prop(e1, src, dst, edge_weight, zeros)
    e3 = _prop(e2, src, dst, edge_weight, zeros)
    users_mean = _users_mean(user_emb, e1, e2, e3, users)
    return _rating(users_mean, item_emb, e1, e2, e3)


# speedup vs baseline: 2.2899x; 2.2899x over previous
import jax
import jax.numpy as jnp
from jax import lax
from jax.experimental import pallas as pl
from jax.experimental.pallas import tpu as pltpu
from jax.experimental.pallas import tpu_sc as plsc

NUM_USERS = 25000
NUM_ITEMS = 25000
N = NUM_USERS + NUM_ITEMS
E = 800000
D = 64
BATCH = 1024

NC = 2
NS = 16
L = 16

HALF = N // NC
K = 80
EPT = E // NS
NCHUNK = EPT // K
ROWS_A = 1568
ROWS_B = HALF - 15 * ROWS_A

_MESH = plsc.VectorSubcoreMesh(
    core_axis_name="c", subcore_axis_name="s", num_cores=NC, num_subcores=NS
)


_SPLAT_DNUMS = lax.GatherDimensionNumbers(
    offset_dims=(), collapsed_slice_dims=(0,), start_index_map=(0,))


def _lane_splat(vec, j):
    idx = jnp.full((L, 1), j, jnp.int32)
    return lax.gather(vec, idx, _SPLAT_DNUMS, (1,),
                      mode=lax.GatherScatterMode.PROMISE_IN_BOUNDS)


def _prop_body(emb_hbm, src_hbm, dst_hbm, w_hbm, zeros_hbm, out_hbm,
               acc, srcbuf, dstbuf, dlbuf, wbuf, rows, sem):
    cid = lax.axis_index("c")
    sid = lax.axis_index("s")
    lo = cid * HALF
    hi = lo + HALF

    @pl.when(sid < NS - 1)
    def _():
        pltpu.sync_copy(zeros_hbm.at[pl.ds(0, ROWS_A)],
                        acc.at[pl.ds(sid * ROWS_A, ROWS_A)])

    @pl.when(sid == NS - 1)
    def _():
        pltpu.sync_copy(zeros_hbm.at[pl.ds(0, ROWS_B)],
                        acc.at[pl.ds(sid * ROWS_A, ROWS_B)])

    plsc.subcore_barrier()

    def chunk_body(i, carry):
        base = sid * EPT + i * K
        pltpu.sync_copy(src_hbm.at[pl.ds(base, K)], srcbuf)
        pltpu.sync_copy(dst_hbm.at[pl.ds(base, K)], dstbuf)
        pltpu.sync_copy(w_hbm.at[pl.ds(base, K)], wbuf)
        pltpu.async_copy(emb_hbm.at[srcbuf], rows, sem).wait()

        for g in range(K // L):
            sl = pl.ds(g * L, L)
            d = dstbuf[sl]
            m = (d >= lo) & (d < hi)
            wv = jnp.where(m, wbuf[sl], 0.0)
            dlbuf[sl] = jnp.where(m, d - lo, 0)
            for j in range(L):
                r = g * L + j
                sw = _lane_splat(wv, j)
                for c in range(D // L):
                    cs = pl.ds(c * L, L)
                    rows[r, cs] = rows[r, cs] * sw
        pltpu.sync_copy(rows, acc.at[dlbuf], add=True)
        return carry

    lax.fori_loop(0, NCHUNK, chunk_body, 0)
    plsc.subcore_barrier()

    @pl.when(sid < NS - 1)
    def _():
        pltpu.sync_copy(acc.at[pl.ds(sid * ROWS_A, ROWS_A)],
                        out_hbm.at[pl.ds(lo + sid * ROWS_A, ROWS_A)])

    @pl.when(sid == NS - 1)
    def _():
        pltpu.sync_copy(acc.at[pl.ds(sid * ROWS_A, ROWS_B)],
                        out_hbm.at[pl.ds(lo + sid * ROWS_A, ROWS_B)])


_prop = pl.kernel(
    _prop_body,
    out_type=jax.ShapeDtypeStruct((N, D), jnp.float32),
    mesh=_MESH,
    scratch_types=[
        pltpu.VMEM_SHARED((HALF, D), jnp.float32),
        pltpu.VMEM((K,), jnp.int32),
        pltpu.VMEM((K,), jnp.int32),
        pltpu.VMEM((K,), jnp.int32),
        pltpu.VMEM((K,), jnp.float32),
        pltpu.VMEM((K, D), jnp.float32),
        pltpu.SemaphoreType.DMA,
    ],
    compiler_params=pltpu.CompilerParams(use_tc_tiling_on_sc=False),
)

UPW = BATCH // (NC * NS)


def _users_body(e0u_hbm, e1_hbm, e2_hbm, e3_hbm, users_hbm, out_hbm,
                idxbuf, b0, b1, b2, b3, sem):
    wid = lax.axis_index("s") * NC + lax.axis_index("c")
    base = wid * UPW
    pltpu.sync_copy(users_hbm.at[pl.ds(base, UPW)], idxbuf)
    pltpu.async_copy(e0u_hbm.at[idxbuf], b0, sem).wait()
    pltpu.async_copy(e1_hbm.at[idxbuf], b1, sem).wait()
    pltpu.async_copy(e2_hbm.at[idxbuf], b2, sem).wait()
    pltpu.async_copy(e3_hbm.at[idxbuf], b3, sem).wait()

    def row_body(r, c2):
        for c in range(D // L):
            cs = pl.ds(c * L, L)
            b0[r, cs] = (b0[r, cs] + b1[r, cs] + b2[r, cs] + b3[r, cs]) * 0.25
        return c2

    lax.fori_loop(0, UPW, row_body, 0, unroll=4)
    pltpu.sync_copy(b0, out_hbm.at[pl.ds(base, UPW)])


_users_mean = pl.kernel(
    _users_body,
    out_type=jax.ShapeDtypeStruct((BATCH, D), jnp.float32),
    mesh=_MESH,
    scratch_types=[
        pltpu.VMEM((UPW,), jnp.int32),
        pltpu.VMEM((UPW, D), jnp.float32),
        pltpu.VMEM((UPW, D), jnp.float32),
        pltpu.VMEM((UPW, D), jnp.float32),
        pltpu.VMEM((UPW, D), jnp.float32),
        pltpu.SemaphoreType.DMA,
    ],
    compiler_params=pltpu.CompilerParams(use_tc_tiling_on_sc=False),
)

IB = 1000
UB = 128


def _items_mean_body(i0_ref, i1_ref, i2_ref, i3_ref, out_ref):
    out_ref[...] = (i0_ref[...] + i1_ref[...] + i2_ref[...]
                    + i3_ref[...]) * 0.25


def _items_mean(item_emb, e1, e2, e3):
    nblk = NUM_ITEMS // IB
    off = NUM_USERS // IB
    return pl.pallas_call(
        _items_mean_body,
        grid=(nblk,),
        in_specs=[
            pl.BlockSpec((IB, D), lambda i: (i, 0)),
            pl.BlockSpec((IB, D), lambda i: (off + i, 0)),
            pl.BlockSpec((IB, D), lambda i: (off + i, 0)),
            pl.BlockSpec((IB, D), lambda i: (off + i, 0)),
        ],
        out_specs=pl.BlockSpec((IB, D), lambda i: (i, 0)),
        out_shape=jax.ShapeDtypeStruct((NUM_ITEMS, D), jnp.float32),
    )(item_emb, e1, e2, e3)


def _rating_body(um_ref, items_ref, out_ref):
    out_ref[...] = lax.dot_general(
        um_ref[...], items_ref[...], (((1,), (1,)), ((), ())),
        preferred_element_type=jnp.float32)


def _rating(users_mean, items_mean):
    return pl.pallas_call(
        _rating_body,
        grid=(BATCH // UB,),
        in_specs=[
            pl.BlockSpec((UB, D), lambda i: (i, 0)),
            pl.BlockSpec((NUM_ITEMS, D), lambda i: (0, 0)),
        ],
        out_specs=pl.BlockSpec((UB, NUM_ITEMS), lambda i: (i, 0)),
        out_shape=jax.ShapeDtypeStruct((BATCH, NUM_ITEMS), jnp.float32),
    )(users_mean, items_mean)


@jax.jit
def kernel(user_emb, item_emb, edge_weight, edge_index, users):
    e0 = jnp.concatenate([user_emb, item_emb], axis=0)
    dst = edge_index[0]
    src = edge_index[1]
    zeros = jnp.zeros((ROWS_A, D), jnp.float32)
    e1 = _prop(e0, src, dst, edge_weight, zeros)
    e2 = _prop(e1, src, dst, edge_weight, zeros)
    e3 = _prop(e2, src, dst, edge_weight, zeros)
    users_mean = _users_mean(user_emb, e1, e2, e3, users)
    items_mean = _items_mean(item_emb, e1, e2, e3)
    return _rating(users_mean, items_mean)


# speedup vs baseline: 4.8113x; 2.1011x over previous
import jax
import jax.numpy as jnp
from jax import lax
from jax.experimental import pallas as pl
from jax.experimental.pallas import tpu as pltpu
from jax.experimental.pallas import tpu_sc as plsc

NUM_USERS = 25000
NUM_ITEMS = 25000
N = NUM_USERS + NUM_ITEMS
E = 800000
D = 64
BATCH = 1024

NC = 2
NS = 16
L = 16

HALF = N // NC
K = 80
EPT = E // NS
NCHUNK = EPT // K
TCHUNK = E // K
SUP = 5
NSUP = NCHUNK // SUP
ROWS_A = 1568
ROWS_B = HALF - 15 * ROWS_A

_MESH = plsc.VectorSubcoreMesh(
    core_axis_name="c", subcore_axis_name="s", num_cores=NC, num_subcores=NS
)


_SPLAT_DNUMS = lax.GatherDimensionNumbers(
    offset_dims=(), collapsed_slice_dims=(0,), start_index_map=(0,))


def _lane_splat(vec, j):
    idx = jnp.full((L, 1), j, jnp.int32)
    return lax.gather(vec, idx, _SPLAT_DNUMS, (1,),
                      mode=lax.GatherScatterMode.PROMISE_IN_BOUNDS)


def _prop_body(emb_hbm, meta_hbm, w_hbm, zeros_hbm, out_hbm,
               acc, mb, wb, rows, dlb, sem_meta, sem_w,
               sg0, sg1, sg2, sg3, sg4):
    cid = lax.axis_index("c")
    sid = lax.axis_index("s")
    lo = cid * HALF
    hi = lo + HALF
    sgs = [sg0, sg1, sg2, sg3, sg4]

    @pl.when(sid < NS - 1)
    def _():
        pltpu.sync_copy(zeros_hbm.at[pl.ds(0, ROWS_A)],
                        acc.at[pl.ds(sid * ROWS_A, ROWS_A)])

    @pl.when(sid == NS - 1)
    def _():
        pltpu.sync_copy(zeros_hbm.at[pl.ds(0, ROWS_B)],
                        acc.at[pl.ds(sid * ROWS_A, ROWS_B)])

    plsc.subcore_barrier()

    c00 = sid * NCHUNK
    pltpu.sync_copy(meta_hbm.at[pl.ds(c00, SUP)], mb.at[0])
    pltpu.sync_copy(w_hbm.at[pl.ds(c00, SUP)], wb.at[0])

    def super_body(i, carry):
        p = lax.rem(i, 2)
        pn = 1 - p
        c0n = jnp.minimum(c00 + (i + 1) * SUP, TCHUNK - SUP)
        pltpu.async_copy(meta_hbm.at[pl.ds(c0n, SUP)], mb.at[pn], sem_meta)
        pltpu.async_copy(w_hbm.at[pl.ds(c0n, SUP)], wb.at[pn], sem_w)
        for b in range(SUP):
            pltpu.async_copy(emb_hbm.at[mb.at[p, b, 0]], rows.at[b], sgs[b])
        for b in range(SUP):
            pltpu.make_async_copy(emb_hbm.at[mb.at[p, b, 0]], rows.at[b],
                                  sgs[b]).wait()
            for g in range(K // L):
                sl = pl.ds(g * L, L)
                d = mb[p, b, 1, sl]
                m = (d >= lo) & (d < hi)
                wv = jnp.where(m, wb[p, b, sl], 0.0)
                dlb[b, sl] = jnp.where(m, d - lo, 0)
                for j in range(L):
                    r = g * L + j
                    sw = _lane_splat(wv, j)
                    for c in range(D // L):
                        cs = pl.ds(c * L, L)
                        rows[b, r, cs] = rows[b, r, cs] * sw
            pltpu.sync_copy(rows.at[b], acc.at[dlb.at[b]], add=True)
        pltpu.make_async_copy(meta_hbm.at[pl.ds(c0n, SUP)], mb.at[pn],
                              sem_meta).wait()
        pltpu.make_async_copy(w_hbm.at[pl.ds(c0n, SUP)], wb.at[pn],
                              sem_w).wait()
        return carry

    lax.fori_loop(0, NSUP, super_body, 0)
    plsc.subcore_barrier()

    @pl.when(sid < NS - 1)
    def _():
        pltpu.sync_copy(acc.at[pl.ds(sid * ROWS_A, ROWS_A)],
                        out_hbm.at[pl.ds(lo + sid * ROWS_A, ROWS_A)])

    @pl.when(sid == NS - 1)
    def _():
        pltpu.sync_copy(acc.at[pl.ds(sid * ROWS_A, ROWS_B)],
                        out_hbm.at[pl.ds(lo + sid * ROWS_A, ROWS_B)])


_prop = pl.kernel(
    _prop_body,
    out_type=jax.ShapeDtypeStruct((N, D), jnp.float32),
    mesh=_MESH,
    scratch_types=[
        pltpu.VMEM_SHARED((HALF, D), jnp.float32),
        pltpu.VMEM((2, SUP, 2, K), jnp.int32),
        pltpu.VMEM((2, SUP, K), jnp.float32),
        pltpu.VMEM((SUP, K, D), jnp.float32),
        pltpu.VMEM((SUP, K), jnp.int32),
        pltpu.SemaphoreType.DMA,
        pltpu.SemaphoreType.DMA,
        pltpu.SemaphoreType.DMA,
        pltpu.SemaphoreType.DMA,
        pltpu.SemaphoreType.DMA,
        pltpu.SemaphoreType.DMA,
        pltpu.SemaphoreType.DMA,
    ],
    compiler_params=pltpu.CompilerParams(use_tc_tiling_on_sc=False),
)

UPW = BATCH // (NC * NS)


def _users_body(e0u_hbm, e1_hbm, e2_hbm, e3_hbm, users_hbm, out_hbm,
                idxbuf, b0, b1, b2, b3, sem):
    wid = lax.axis_index("s") * NC + lax.axis_index("c")
    base = wid * UPW
    pltpu.sync_copy(users_hbm.at[pl.ds(base, UPW)], idxbuf)
    pltpu.async_copy(e0u_hbm.at[idxbuf], b0, sem).wait()
    pltpu.async_copy(e1_hbm.at[idxbuf], b1, sem).wait()
    pltpu.async_copy(e2_hbm.at[idxbuf], b2, sem).wait()
    pltpu.async_copy(e3_hbm.at[idxbuf], b3, sem).wait()

    def row_body(r, c2):
        for c in range(D // L):
            cs = pl.ds(c * L, L)
            b0[r, cs] = (b0[r, cs] + b1[r, cs] + b2[r, cs] + b3[r, cs]) * 0.25
        return c2

    lax.fori_loop(0, UPW, row_body, 0, unroll=4)
    pltpu.sync_copy(b0, out_hbm.at[pl.ds(base, UPW)])


_users_mean = pl.kernel(
    _users_body,
    out_type=jax.ShapeDtypeStruct((BATCH, D), jnp.float32),
    mesh=_MESH,
    scratch_types=[
        pltpu.VMEM((UPW,), jnp.int32),
        pltpu.VMEM((UPW, D), jnp.float32),
        pltpu.VMEM((UPW, D), jnp.float32),
        pltpu.VMEM((UPW, D), jnp.float32),
        pltpu.VMEM((UPW, D), jnp.float32),
        pltpu.SemaphoreType.DMA,
    ],
    compiler_params=pltpu.CompilerParams(use_tc_tiling_on_sc=False),
)

IB = 1000
UB = 128


def _items_mean_body(i0_ref, i1_ref, i2_ref, i3_ref, out_ref):
    out_ref[...] = (i0_ref[...] + i1_ref[...] + i2_ref[...]
                    + i3_ref[...]) * 0.25


def _items_mean(item_emb, e1, e2, e3):
    nblk = NUM_ITEMS // IB
    off = NUM_USERS // IB
    return pl.pallas_call(
        _items_mean_body,
        grid=(nblk,),
        in_specs=[
            pl.BlockSpec((IB, D), lambda i: (i, 0)),
            pl.BlockSpec((IB, D), lambda i: (off + i, 0)),
            pl.BlockSpec((IB, D), lambda i: (off + i, 0)),
            pl.BlockSpec((IB, D), lambda i: (off + i, 0)),
        ],
        out_specs=pl.BlockSpec((IB, D), lambda i: (i, 0)),
        out_shape=jax.ShapeDtypeStruct((NUM_ITEMS, D), jnp.float32),
    )(item_emb, e1, e2, e3)


def _rating_body(um_ref, items_ref, out_ref):
    out_ref[...] = lax.dot_general(
        um_ref[...], items_ref[...], (((1,), (1,)), ((), ())),
        preferred_element_type=jnp.float32)


def _rating(users_mean, items_mean):
    return pl.pallas_call(
        _rating_body,
        grid=(BATCH // UB,),
        in_specs=[
            pl.BlockSpec((UB, D), lambda i: (i, 0)),
            pl.BlockSpec((NUM_ITEMS, D), lambda i: (0, 0)),
        ],
        out_specs=pl.BlockSpec((UB, NUM_ITEMS), lambda i: (i, 0)),
        out_shape=jax.ShapeDtypeStruct((BATCH, NUM_ITEMS), jnp.float32),
    )(users_mean, items_mean)


@jax.jit
def kernel(user_emb, item_emb, edge_weight, edge_index, users):
    e0 = jnp.concatenate([user_emb, item_emb], axis=0)
    meta = (jnp.stack([edge_index[1], edge_index[0]], axis=0)
            .reshape(2, TCHUNK, K).transpose(1, 0, 2))
    wchunk = edge_weight.reshape(TCHUNK, K)
    zeros = jnp.zeros((ROWS_A, D), jnp.float32)
    e1 = _prop(e0, meta, wchunk, zeros)
    e2 = _prop(e1, meta, wchunk, zeros)
    e3 = _prop(e2, meta, wchunk, zeros)
    users_mean = _users_mean(user_emb, e1, e2, e3, users)
    items_mean = _items_mean(item_emb, e1, e2, e3)
    return _rating(users_mean, items_mean)


# speedup vs baseline: 5.8191x; 1.2095x over previous
import jax
import jax.numpy as jnp
from jax import lax
from jax.experimental import pallas as pl
from jax.experimental.pallas import tpu as pltpu
from jax.experimental.pallas import tpu_sc as plsc

NUM_USERS = 25000
NUM_ITEMS = 25000
N = NUM_USERS + NUM_ITEMS
E = 800000
D = 64
BATCH = 1024

NC = 2
NS = 16
L = 16

HALF = N // NC
K = 80
EPT = E // NS
NCHUNK = EPT // K
TCHUNK = E // K
SUP = 5
NSUP = NCHUNK // SUP
ROWS_A = 1568
ROWS_B = HALF - 15 * ROWS_A

_MESH = plsc.VectorSubcoreMesh(
    core_axis_name="c", subcore_axis_name="s", num_cores=NC, num_subcores=NS
)


_SPLAT_DNUMS = lax.GatherDimensionNumbers(
    offset_dims=(), collapsed_slice_dims=(0,), start_index_map=(0,))


def _lane_splat(vec, j):
    idx = jnp.full((L, 1), j, jnp.int32)
    return lax.gather(vec, idx, _SPLAT_DNUMS, (1,),
                      mode=lax.GatherScatterMode.PROMISE_IN_BOUNDS)


def _prop_body(emb_hbm, meta_hbm, w_hbm, zeros_hbm, out_hbm,
               acc, mslot, wslot, rows, dlb,
               sm0, sm1, sm2, sm3, sw0, sw1, sw2, sw3,
               sg0, sg1, sg2, sg3):
    cid = lax.axis_index("c")
    sid = lax.axis_index("s")
    lo = cid * HALF
    hi = lo + HALF
    sms = [sm0, sm1, sm2, sm3]
    sws = [sw0, sw1, sw2, sw3]
    sgs = [sg0, sg1, sg2, sg3]

    @pl.when(sid < NS - 1)
    def _():
        pltpu.sync_copy(zeros_hbm.at[pl.ds(0, ROWS_A)],
                        acc.at[pl.ds(sid * ROWS_A, ROWS_A)])

    @pl.when(sid == NS - 1)
    def _():
        pltpu.sync_copy(zeros_hbm.at[pl.ds(0, ROWS_B)],
                        acc.at[pl.ds(sid * ROWS_A, ROWS_B)])

    plsc.subcore_barrier()

    c00 = sid * NCHUNK

    def cidx(k):
        return c00 + jnp.minimum(k, NCHUNK - 1)

    def meta_issue(k, u):
        pltpu.async_copy(meta_hbm.at[cidx(k)], mslot.at[u], sms[u])
        pltpu.async_copy(w_hbm.at[cidx(k)], wslot.at[u], sws[u])

    def meta_wait(k, u):
        pltpu.make_async_copy(meta_hbm.at[cidx(k)], mslot.at[u],
                              sms[u]).wait()
        pltpu.make_async_copy(w_hbm.at[cidx(k)], wslot.at[u],
                              sws[u]).wait()

    def gather_issue(k, u):
        pltpu.async_copy(emb_hbm.at[mslot.at[u, 0]], rows.at[u], sgs[u])

    def gather_wait(k, u):
        pltpu.make_async_copy(emb_hbm.at[mslot.at[u, 0]], rows.at[u],
                              sgs[u]).wait()

    def compute_scatter(u):
        for g in range(K // L):
            sl = pl.ds(g * L, L)
            d = mslot[u, 1, sl]
            m = (d >= lo) & (d < hi)
            wv = jnp.where(m, wslot[u, sl], 0.0)
            dlb[sl] = jnp.where(m, d - lo, 0)
            for j in range(L):
                r = g * L + j
                sw = _lane_splat(wv, j)
                for c in range(D // L):
                    cs = pl.ds(c * L, L)
                    rows[u, r, cs] = rows[u, r, cs] * sw
        pltpu.sync_copy(rows.at[u], acc.at[dlb], add=True)

    pltpu.sync_copy(meta_hbm.at[cidx(0)], mslot.at[0])
    pltpu.sync_copy(w_hbm.at[cidx(0)], wslot.at[0])
    pltpu.sync_copy(meta_hbm.at[cidx(1)], mslot.at[1])
    pltpu.sync_copy(w_hbm.at[cidx(1)], wslot.at[1])
    meta_issue(2, 2)
    meta_issue(3, 3)
    gather_issue(0, 0)
    gather_issue(1, 1)

    def quad_body(jj, carry):
        k0 = jj * 4
        for u in range(4):
            k = k0 + u
            u2 = (u + 2) % 4
            meta_wait(k + 2, u2)
            gather_issue(k + 2, u2)
            gather_wait(k, u)
            compute_scatter(u)
            meta_issue(k + 4, u)
        return carry

    lax.fori_loop(0, (NCHUNK - 1) // 4, quad_body, 0)

    gather_wait(NCHUNK - 1, 0)
    compute_scatter(0)

    meta_wait(NCHUNK + 1, 2)
    meta_wait(NCHUNK + 2, 3)
    gather_wait(NCHUNK, 1)
    plsc.subcore_barrier()

    @pl.when(sid < NS - 1)
    def _():
        pltpu.sync_copy(acc.at[pl.ds(sid * ROWS_A, ROWS_A)],
                        out_hbm.at[pl.ds(lo + sid * ROWS_A, ROWS_A)])

    @pl.when(sid == NS - 1)
    def _():
        pltpu.sync_copy(acc.at[pl.ds(sid * ROWS_A, ROWS_B)],
                        out_hbm.at[pl.ds(lo + sid * ROWS_A, ROWS_B)])


_prop = pl.kernel(
    _prop_body,
    out_type=jax.ShapeDtypeStruct((N, D), jnp.float32),
    mesh=_MESH,
    scratch_types=[
        pltpu.VMEM_SHARED((HALF, D), jnp.float32),
        pltpu.VMEM((4, 2, K), jnp.int32),
        pltpu.VMEM((4, K), jnp.float32),
        pltpu.VMEM((4, K, D), jnp.float32),
        pltpu.VMEM((K,), jnp.int32),
    ] + [pltpu.SemaphoreType.DMA] * 12,
    compiler_params=pltpu.CompilerParams(use_tc_tiling_on_sc=False),
)

UPW = BATCH // (NC * NS)


def _users_body(e0u_hbm, e1_hbm, e2_hbm, e3_hbm, users_hbm, out_hbm,
                idxbuf, b0, b1, b2, b3, sem):
    wid = lax.axis_index("s") * NC + lax.axis_index("c")
    base = wid * UPW
    pltpu.sync_copy(users_hbm.at[pl.ds(base, UPW)], idxbuf)
    pltpu.async_copy(e0u_hbm.at[idxbuf], b0, sem).wait()
    pltpu.async_copy(e1_hbm.at[idxbuf], b1, sem).wait()
    pltpu.async_copy(e2_hbm.at[idxbuf], b2, sem).wait()
    pltpu.async_copy(e3_hbm.at[idxbuf], b3, sem).wait()

    def row_body(r, c2):
        for c in range(D // L):
            cs = pl.ds(c * L, L)
            b0[r, cs] = (b0[r, cs] + b1[r, cs] + b2[r, cs] + b3[r, cs]) * 0.25
        return c2

    lax.fori_loop(0, UPW, row_body, 0, unroll=4)
    pltpu.sync_copy(b0, out_hbm.at[pl.ds(base, UPW)])


_users_mean = pl.kernel(
    _users_body,
    out_type=jax.ShapeDtypeStruct((BATCH, D), jnp.float32),
    mesh=_MESH,
    scratch_types=[
        pltpu.VMEM((UPW,), jnp.int32),
        pltpu.VMEM((UPW, D), jnp.float32),
        pltpu.VMEM((UPW, D), jnp.float32),
        pltpu.VMEM((UPW, D), jnp.float32),
        pltpu.VMEM((UPW, D), jnp.float32),
        pltpu.SemaphoreType.DMA,
    ],
    compiler_params=pltpu.CompilerParams(use_tc_tiling_on_sc=False),
)

IB = 1000
UB = 128


def _items_mean_body(i0_ref, i1_ref, i2_ref, i3_ref, out_ref):
    out_ref[...] = (i0_ref[...] + i1_ref[...] + i2_ref[...]
                    + i3_ref[...]) * 0.25


def _items_mean(item_emb, e1, e2, e3):
    nblk = NUM_ITEMS // IB
    off = NUM_USERS // IB
    return pl.pallas_call(
        _items_mean_body,
        grid=(nblk,),
        in_specs=[
            pl.BlockSpec((IB, D), lambda i: (i, 0)),
            pl.BlockSpec((IB, D), lambda i: (off + i, 0)),
            pl.BlockSpec((IB, D), lambda i: (off + i, 0)),
            pl.BlockSpec((IB, D), lambda i: (off + i, 0)),
        ],
        out_specs=pl.BlockSpec((IB, D), lambda i: (i, 0)),
        out_shape=jax.ShapeDtypeStruct((NUM_ITEMS, D), jnp.float32),
    )(item_emb, e1, e2, e3)


def _rating_body(um_ref, items_ref, out_ref):
    out_ref[...] = lax.dot_general(
        um_ref[...], items_ref[...], (((1,), (1,)), ((), ())),
        preferred_element_type=jnp.float32)


def _rating(users_mean, items_mean):
    return pl.pallas_call(
        _rating_body,
        grid=(BATCH // UB,),
        in_specs=[
            pl.BlockSpec((UB, D), lambda i: (i, 0)),
            pl.BlockSpec((NUM_ITEMS, D), lambda i: (0, 0)),
        ],
        out_specs=pl.BlockSpec((UB, NUM_ITEMS), lambda i: (i, 0)),
        out_shape=jax.ShapeDtypeStruct((BATCH, NUM_ITEMS), jnp.float32),
    )(users_mean, items_mean)


@jax.jit
def kernel(user_emb, item_emb, edge_weight, edge_index, users):
    e0 = jnp.concatenate([user_emb, item_emb], axis=0)
    meta = (jnp.stack([edge_index[1], edge_index[0]], axis=0)
            .reshape(2, TCHUNK, K).transpose(1, 0, 2))
    wchunk = edge_weight.reshape(TCHUNK, K)
    zeros = jnp.zeros((ROWS_A, D), jnp.float32)
    e1 = _prop(e0, meta, wchunk, zeros)
    e2 = _prop(e1, meta, wchunk, zeros)
    e3 = _prop(e2, meta, wchunk, zeros)
    users_mean = _users_mean(user_emb, e1, e2, e3, users)
    items_mean = _items_mean(item_emb, e1, e2, e3)
    return _rating(users_mean, items_mean)


# speedup vs baseline: 9.4358x; 1.6215x over previous
import jax
import jax.numpy as jnp
from jax import lax
from jax.experimental import pallas as pl
from jax.experimental.pallas import tpu as pltpu
from jax.experimental.pallas import tpu_sc as plsc

NUM_USERS = 25000
NUM_ITEMS = 25000
N = NUM_USERS + NUM_ITEMS
E = 800000
D = 64
BATCH = 1024

NC = 2
NS = 16
L = 16

DH = D // NC
K = 80
NCHUNK = (E // NS) // K
ROWS_A = 3128
ROWS_B = N - 15 * ROWS_A

_MESH = plsc.VectorSubcoreMesh(
    core_axis_name="c", subcore_axis_name="s", num_cores=NC, num_subcores=NS
)

_SPLAT_DNUMS = lax.GatherDimensionNumbers(
    offset_dims=(), collapsed_slice_dims=(0,), start_index_map=(0,))


def _lane_splat(vec, j):
    idx = jnp.full((L, 1), j, jnp.int32)
    return lax.gather(vec, idx, _SPLAT_DNUMS, (1,),
                      mode=lax.GatherScatterMode.PROMISE_IN_BOUNDS)


def _prop_body(emb_hbm, meta_hbm, w_hbm, zeros_hbm, out_hbm,
               acc, mslot, wslot, rows, sm0, sm1, sm2, sm3,
               sw0, sw1, sw2, sw3, sg0, sg1, sg2, sg3):
    cid = lax.axis_index("c")
    sid = lax.axis_index("s")
    sms = [sm0, sm1, sm2, sm3]
    sws = [sw0, sw1, sw2, sw3]
    sgs = [sg0, sg1, sg2, sg3]
    emb_h = emb_hbm.at[cid]

    @pl.when(sid < NS - 1)
    def _():
        pltpu.sync_copy(zeros_hbm.at[pl.ds(0, ROWS_A)],
                        acc.at[pl.ds(sid * ROWS_A, ROWS_A)])

    @pl.when(sid == NS - 1)
    def _():
        pltpu.sync_copy(zeros_hbm.at[pl.ds(0, ROWS_B)],
                        acc.at[pl.ds(sid * ROWS_A, ROWS_B)])

    plsc.subcore_barrier()

    c00 = sid * NCHUNK

    def cidx(k):
        return c00 + jnp.minimum(k, NCHUNK - 1)

    def meta_issue(k, u):
        pltpu.async_copy(meta_hbm.at[cidx(k)], mslot.at[u], sms[u])
        pltpu.async_copy(w_hbm.at[cidx(k)], wslot.at[u], sws[u])

    def meta_wait(k, u):
        pltpu.make_async_copy(meta_hbm.at[cidx(k)], mslot.at[u],
                              sms[u]).wait()
        pltpu.make_async_copy(w_hbm.at[cidx(k)], wslot.at[u],
                              sws[u]).wait()

    def gather_issue(k, u):
        pltpu.async_copy(emb_h.at[mslot.at[u, 0]], rows.at[u], sgs[u])

    def gather_wait(k, u):
        pltpu.make_async_copy(emb_h.at[mslot.at[u, 0]], rows.at[u],
                              sgs[u]).wait()

    def compute_scatter(u):
        for g in range(K // L):
            sl = pl.ds(g * L, L)
            wv = wslot[u, sl]
            for j in range(L):
                r = g * L + j
                sw = _lane_splat(wv, j)
                for c in range(DH // L):
                    cs = pl.ds(c * L, L)
                    rows[u, r, cs] = rows[u, r, cs] * sw
        pltpu.sync_copy(rows.at[u], acc.at[mslot.at[u, 1]], add=True)

    pltpu.sync_copy(meta_hbm.at[cidx(0)], mslot.at[0])
    pltpu.sync_copy(w_hbm.at[cidx(0)], wslot.at[0])
    pltpu.sync_copy(meta_hbm.at[cidx(1)], mslot.at[1])
    pltpu.sync_copy(w_hbm.at[cidx(1)], wslot.at[1])
    meta_issue(2, 2)
    meta_issue(3, 3)
    gather_issue(0, 0)
    gather_issue(1, 1)

    def quad_body(jj, carry):
        k0 = jj * 4
        for u in range(4):
            k = k0 + u
            u2 = (u + 2) % 4
            meta_wait(k + 2, u2)
            gather_issue(k + 2, u2)
            gather_wait(k, u)
            compute_scatter(u)
            meta_issue(k + 4, u)
        return carry

    lax.fori_loop(0, (NCHUNK - 1) // 4, quad_body, 0)

    gather_wait(NCHUNK - 1, 0)
    compute_scatter(0)

    meta_wait(NCHUNK + 1, 2)
    meta_wait(NCHUNK + 2, 3)
    gather_wait(NCHUNK, 1)
    plsc.subcore_barrier()

    outc = out_hbm.at[cid]

    @pl.when(sid < NS - 1)
    def _():
        pltpu.sync_copy(acc.at[pl.ds(sid * ROWS_A, ROWS_A)],
                        outc.at[pl.ds(sid * ROWS_A, ROWS_A)])

    @pl.when(sid == NS - 1)
    def _():
        pltpu.sync_copy(acc.at[pl.ds(sid * ROWS_A, ROWS_B)],
                        outc.at[pl.ds(sid * ROWS_A, ROWS_B)])


_prop = pl.kernel(
    _prop_body,
    out_type=jax.ShapeDtypeStruct((NC, N, DH), jnp.float32),
    mesh=_MESH,
    scratch_types=[
        pltpu.VMEM_SHARED((N, DH), jnp.float32),
        pltpu.VMEM((4, 2, K), jnp.int32),
        pltpu.VMEM((4, K), jnp.float32),
        pltpu.VMEM((4, K, DH), jnp.float32),
    ] + [pltpu.SemaphoreType.DMA] * 12,
    compiler_params=pltpu.CompilerParams(use_tc_tiling_on_sc=False),
)

UPW = BATCH // (NC * NS)


def _users_body(e0_hbm, e1_hbm, e2_hbm, e3_hbm, users_hbm, out_hbm,
                idxbuf, b0, b1, b2, b3, ob, sem):
    wid = lax.axis_index("s") * NC + lax.axis_index("c")
    base = wid * UPW
    pltpu.sync_copy(users_hbm.at[pl.ds(base, UPW)], idxbuf)
    for h in range(NC):
        for tbl, buf in ((e0_hbm, b0), (e1_hbm, b1),
                         (e2_hbm, b2), (e3_hbm, b3)):
            pltpu.async_copy(tbl.at[h].at[idxbuf], buf, sem).wait()

        def row_body(r, carry, h=h):
            for c in range(DH // L):
                cs = pl.ds(c * L, L)
                os = pl.ds(h * DH + c * L, L)
                ob[r, os] = (b0[r, cs] + b1[r, cs]
                             + b2[r, cs] + b3[r, cs]) * 0.25
            return carry

        lax.fori_loop(0, UPW, row_body, 0, unroll=4)
    pltpu.sync_copy(ob, out_hbm.at[pl.ds(base, UPW)])


_users_mean = pl.kernel(
    _users_body,
    out_type=jax.ShapeDtypeStruct((BATCH, D), jnp.float32),
    mesh=_MESH,
    scratch_types=[
        pltpu.VMEM((UPW,), jnp.int32),
        pltpu.VMEM((UPW, DH), jnp.float32),
        pltpu.VMEM((UPW, DH), jnp.float32),
        pltpu.VMEM((UPW, DH), jnp.float32),
        pltpu.VMEM((UPW, DH), jnp.float32),
        pltpu.VMEM((UPW, D), jnp.float32),
        pltpu.SemaphoreType.DMA,
    ],
    compiler_params=pltpu.CompilerParams(use_tc_tiling_on_sc=False),
)

IB = 1000
UB = 128


def _items_mean_body(a0, a1, b0, b1, c0, c1, d0, d1, out_ref):
    left = (a0[...] + b0[...] + c0[...] + d0[...]) * 0.25
    right = (a1[...] + b1[...] + c1[...] + d1[...]) * 0.25
    out_ref[...] = jnp.concatenate([left, right], axis=1)


def _items_mean(e0, e1, e2, e3):
    nblk = NUM_ITEMS // IB
    off = NUM_USERS // IB
    in_specs = []
    for _ in range(4):
        in_specs.extend(
            pl.BlockSpec((None, IB, DH), lambda i, h=h: (h, off + i, 0))
            for h in range(NC))
    return pl.pallas_call(
        _items_mean_body,
        grid=(nblk,),
        in_specs=in_specs,
        out_specs=pl.BlockSpec((IB, D), lambda i: (i, 0)),
        out_shape=jax.ShapeDtypeStruct((NUM_ITEMS, D), jnp.float32),
    )(e0, e0, e1, e1, e2, e2, e3, e3)


def _rating_body(um_ref, items_ref, out_ref):
    out_ref[...] = lax.dot_general(
        um_ref[...], items_ref[...], (((1,), (1,)), ((), ())),
        preferred_element_type=jnp.float32)


def _rating(users_mean, items_mean):
    return pl.pallas_call(
        _rating_body,
        grid=(BATCH // UB,),
        in_specs=[
            pl.BlockSpec((UB, D), lambda i: (i, 0)),
            pl.BlockSpec((NUM_ITEMS, D), lambda i: (0, 0)),
        ],
        out_specs=pl.BlockSpec((UB, NUM_ITEMS), lambda i: (i, 0)),
        out_shape=jax.ShapeDtypeStruct((BATCH, NUM_ITEMS), jnp.float32),
    )(users_mean, items_mean)


@jax.jit
def kernel(user_emb, item_emb, edge_weight, edge_index, users):
    e0 = jnp.concatenate([user_emb, item_emb], axis=0)
    e0s = jnp.stack([e0[:, :DH], e0[:, DH:]], axis=0)
    tchunk = E // K
    meta = (jnp.stack([edge_index[1], edge_index[0]], axis=0)
            .reshape(2, tchunk, K).transpose(1, 0, 2))
    wchunk = edge_weight.reshape(tchunk, K)
    zeros = jnp.zeros((ROWS_A, DH), jnp.float32)
    e1s = _prop(e0s, meta, wchunk, zeros)
    e2s = _prop(e1s, meta, wchunk, zeros)
    e3s = _prop(e2s, meta, wchunk, zeros)
    users_mean = _users_mean(e0s, e1s, e2s, e3s, users)
    items_mean = _items_mean(e0s, e1s, e2s, e3s)
    return _rating(users_mean, items_mean)


# speedup vs baseline: 10.1241x; 1.0730x over previous
import jax
import jax.numpy as jnp
from jax import lax
from jax.experimental import pallas as pl
from jax.experimental.pallas import tpu as pltpu
from jax.experimental.pallas import tpu_sc as plsc

NUM_USERS = 25000
NUM_ITEMS = 25000
N = NUM_USERS + NUM_ITEMS
E = 800000
D = 64
BATCH = 1024

NC = 2
NS = 16
L = 16

DH = D // NC
K = 80
NCHUNK = (E // NS) // K
ROWS_A = 3128
ROWS_B = N - 15 * ROWS_A

_MESH = plsc.VectorSubcoreMesh(
    core_axis_name="c", subcore_axis_name="s", num_cores=NC, num_subcores=NS
)

_SPLAT_DNUMS = lax.GatherDimensionNumbers(
    offset_dims=(), collapsed_slice_dims=(0,), start_index_map=(0,))


def _lane_splat(vec, j):
    idx = jnp.full((L, 1), j, jnp.int32)
    return lax.gather(vec, idx, _SPLAT_DNUMS, (1,),
                      mode=lax.GatherScatterMode.PROMISE_IN_BOUNDS)


def _prop_body(emb_hbm, meta_hbm, w_hbm, zeros_hbm, out_hbm,
               acc, mslot, wslot, dslot, rows, sm0, sm1, sm2, sm3,
               sw0, sw1, sw2, sw3, sg0, sg1, sg2, sg3,
               sc0, sc1, sc2, sc3):
    cid = lax.axis_index("c")
    sid = lax.axis_index("s")
    sms = [sm0, sm1, sm2, sm3]
    sws = [sw0, sw1, sw2, sw3]
    sgs = [sg0, sg1, sg2, sg3]
    scs = [sc0, sc1, sc2, sc3]
    emb_h = emb_hbm.at[cid]

    @pl.when(sid < NS - 1)
    def _():
        pltpu.sync_copy(zeros_hbm.at[pl.ds(0, ROWS_A)],
                        acc.at[pl.ds(sid * ROWS_A, ROWS_A)])

    @pl.when(sid == NS - 1)
    def _():
        pltpu.sync_copy(zeros_hbm.at[pl.ds(0, ROWS_B)],
                        acc.at[pl.ds(sid * ROWS_A, ROWS_B)])

    plsc.subcore_barrier()

    c00 = sid * NCHUNK

    def cidx(k):
        return c00 + jnp.minimum(k, NCHUNK - 1)

    def meta_issue(k, u):
        pltpu.async_copy(meta_hbm.at[cidx(k)], mslot.at[u], sms[u])
        pltpu.async_copy(w_hbm.at[cidx(k)], wslot.at[u], sws[u])

    def meta_wait(k, u):
        pltpu.make_async_copy(meta_hbm.at[cidx(k)], mslot.at[u],
                              sms[u]).wait()
        pltpu.make_async_copy(w_hbm.at[cidx(k)], wslot.at[u],
                              sws[u]).wait()

    def gather_issue(k, u):
        pltpu.async_copy(emb_h.at[mslot.at[u, 0]], rows.at[u], sgs[u])

    def gather_wait(k, u):
        pltpu.make_async_copy(emb_h.at[mslot.at[u, 0]], rows.at[u],
                              sgs[u]).wait()

    def compute_scatter(u):
        for g in range(K // L):
            sl = pl.ds(g * L, L)
            dslot[u, sl] = mslot[u, 1, sl]
            wv = wslot[u, sl]
            for j in range(L):
                r = g * L + j
                sw = _lane_splat(wv, j)
                for c in range(DH // L):
                    cs = pl.ds(c * L, L)
                    rows[u, r, cs] = rows[u, r, cs] * sw
        pltpu.async_copy(rows.at[u], acc.at[dslot.at[u]], scs[u], add=True)

    def scatter_wait(u):
        pltpu.make_async_copy(rows.at[u], acc.at[dslot.at[u]],
                              scs[u]).wait()

    pltpu.sync_copy(meta_hbm.at[cidx(0)], mslot.at[0])
    pltpu.sync_copy(w_hbm.at[cidx(0)], wslot.at[0])
    pltpu.sync_copy(meta_hbm.at[cidx(1)], mslot.at[1])
    pltpu.sync_copy(w_hbm.at[cidx(1)], wslot.at[1])
    meta_issue(2, 2)
    meta_issue(3, 3)
    gather_issue(0, 0)
    gather_issue(1, 1)

    def quad_body(jj, carry):
        k0 = jj * 4
        for u in range(4):
            k = k0 + u
            u2 = (u + 2) % 4
            meta_wait(k + 2, u2)

            @pl.when(k >= 2)
            def _():
                scatter_wait(u2)

            gather_issue(k + 2, u2)
            gather_wait(k, u)
            compute_scatter(u)
            meta_issue(k + 4, u)
        return carry

    lax.fori_loop(0, (NCHUNK - 1) // 4, quad_body, 0)

    gather_wait(NCHUNK - 1, 0)
    compute_scatter(0)

    meta_wait(NCHUNK + 1, 2)
    meta_wait(NCHUNK + 2, 3)
    gather_wait(NCHUNK, 1)
    scatter_wait(2)
    scatter_wait(3)
    scatter_wait(0)
    plsc.subcore_barrier()

    outc = out_hbm.at[cid]

    @pl.when(sid < NS - 1)
    def _():
        pltpu.sync_copy(acc.at[pl.ds(sid * ROWS_A, ROWS_A)],
                        outc.at[pl.ds(sid * ROWS_A, ROWS_A)])

    @pl.when(sid == NS - 1)
    def _():
        pltpu.sync_copy(acc.at[pl.ds(sid * ROWS_A, ROWS_B)],
                        outc.at[pl.ds(sid * ROWS_A, ROWS_B)])


_prop = pl.kernel(
    _prop_body,
    out_type=jax.ShapeDtypeStruct((NC, N, DH), jnp.float32),
    mesh=_MESH,
    scratch_types=[
        pltpu.VMEM_SHARED((N, DH), jnp.float32),
        pltpu.VMEM((4, 2, K), jnp.int32),
        pltpu.VMEM((4, K), jnp.float32),
        pltpu.VMEM((4, K), jnp.int32),
        pltpu.VMEM((4, K, DH), jnp.float32),
    ] + [pltpu.SemaphoreType.DMA] * 16,
    compiler_params=pltpu.CompilerParams(use_tc_tiling_on_sc=False),
)

UPW = BATCH // (NC * NS)


def _users_body(e0_hbm, e1_hbm, e2_hbm, e3_hbm, users_hbm, out_hbm,
                idxbuf, b0, b1, b2, b3, ob, sem):
    wid = lax.axis_index("s") * NC + lax.axis_index("c")
    base = wid * UPW
    pltpu.sync_copy(users_hbm.at[pl.ds(base, UPW)], idxbuf)
    for h in range(NC):
        for tbl, buf in ((e0_hbm, b0), (e1_hbm, b1),
                         (e2_hbm, b2), (e3_hbm, b3)):
            pltpu.async_copy(tbl.at[h].at[idxbuf], buf, sem).wait()

        def row_body(r, carry, h=h):
            for c in range(DH // L):
                cs = pl.ds(c * L, L)
                os = pl.ds(h * DH + c * L, L)
                ob[r, os] = (b0[r, cs] + b1[r, cs]
                             + b2[r, cs] + b3[r, cs]) * 0.25
            return carry

        lax.fori_loop(0, UPW, row_body, 0, unroll=4)
    pltpu.sync_copy(ob, out_hbm.at[pl.ds(base, UPW)])


_users_mean = pl.kernel(
    _users_body,
    out_type=jax.ShapeDtypeStruct((BATCH, D), jnp.float32),
    mesh=_MESH,
    scratch_types=[
        pltpu.VMEM((UPW,), jnp.int32),
        pltpu.VMEM((UPW, DH), jnp.float32),
        pltpu.VMEM((UPW, DH), jnp.float32),
        pltpu.VMEM((UPW, DH), jnp.float32),
        pltpu.VMEM((UPW, DH), jnp.float32),
        pltpu.VMEM((UPW, D), jnp.float32),
        pltpu.SemaphoreType.DMA,
    ],
    compiler_params=pltpu.CompilerParams(use_tc_tiling_on_sc=False),
)

IB = 1000
UB = 128


def _items_mean_body(a0, a1, b0, b1, c0, c1, d0, d1, out_ref):
    left = (a0[...] + b0[...] + c0[...] + d0[...]) * 0.25
    right = (a1[...] + b1[...] + c1[...] + d1[...]) * 0.25
    out_ref[...] = jnp.concatenate([left, right], axis=1)


def _items_mean(e0, e1, e2, e3):
    nblk = NUM_ITEMS // IB
    off = NUM_USERS // IB
    in_specs = []
    for _ in range(4):
        in_specs.extend(
            pl.BlockSpec((None, IB, DH), lambda i, h=h: (h, off + i, 0))
            for h in range(NC))
    return pl.pallas_call(
        _items_mean_body,
        grid=(nblk,),
        in_specs=in_specs,
        out_specs=pl.BlockSpec((IB, D), lambda i: (i, 0)),
        out_shape=jax.ShapeDtypeStruct((NUM_ITEMS, D), jnp.float32),
    )(e0, e0, e1, e1, e2, e2, e3, e3)


def _rating_body(um_ref, items_ref, out_ref):
    out_ref[...] = lax.dot_general(
        um_ref[...], items_ref[...], (((1,), (1,)), ((), ())),
        preferred_element_type=jnp.float32)


def _rating(users_mean, items_mean):
    return pl.pallas_call(
        _rating_body,
        grid=(BATCH // UB,),
        in_specs=[
            pl.BlockSpec((UB, D), lambda i: (i, 0)),
            pl.BlockSpec((NUM_ITEMS, D), lambda i: (0, 0)),
        ],
        out_specs=pl.BlockSpec((UB, NUM_ITEMS), lambda i: (i, 0)),
        out_shape=jax.ShapeDtypeStruct((BATCH, NUM_ITEMS), jnp.float32),
    )(users_mean, items_mean)


@jax.jit
def kernel(user_emb, item_emb, edge_weight, edge_index, users):
    e0 = jnp.concatenate([user_emb, item_emb], axis=0)
    e0s = jnp.stack([e0[:, :DH], e0[:, DH:]], axis=0)
    tchunk = E // K
    meta = (jnp.stack([edge_index[1], edge_index[0]], axis=0)
            .reshape(2, tchunk, K).transpose(1, 0, 2))
    wchunk = edge_weight.reshape(tchunk, K)
    zeros = jnp.zeros((ROWS_A, DH), jnp.float32)
    e1s = _prop(e0s, meta, wchunk, zeros)
    e2s = _prop(e1s, meta, wchunk, zeros)
    e3s = _prop(e2s, meta, wchunk, zeros)
    users_mean = _users_mean(e0s, e1s, e2s, e3s, users)
    items_mean = _items_mean(e0s, e1s, e2s, e3s)
    return _rating(users_mean, items_mean)


# speedup vs baseline: 10.5970x; 1.0467x over previous
import jax
import jax.numpy as jnp
from jax import lax
from jax.experimental import pallas as pl
from jax.experimental.pallas import tpu as pltpu
from jax.experimental.pallas import tpu_sc as plsc

NUM_USERS = 25000
NUM_ITEMS = 25000
N = NUM_USERS + NUM_ITEMS
E = 800000
D = 64
BATCH = 1024

NC = 2
NS = 16
L = 16

DH = D // NC
K = 80
NCHUNK = (E // NS) // K
ROWS_A = 3128
ROWS_B = N - 15 * ROWS_A

_MESH = plsc.VectorSubcoreMesh(
    core_axis_name="c", subcore_axis_name="s", num_cores=NC, num_subcores=NS
)

_SPLAT_DNUMS = lax.GatherDimensionNumbers(
    offset_dims=(), collapsed_slice_dims=(0,), start_index_map=(0,))


def _lane_splat(vec, j):
    idx = jnp.full((L, 1), j, jnp.int32)
    return lax.gather(vec, idx, _SPLAT_DNUMS, (1,),
                      mode=lax.GatherScatterMode.PROMISE_IN_BOUNDS)


def _prop_body(emb_hbm, src_hbm, dst_hbm, w_hbm, zeros_hbm, out_hbm,
               acc, sslot, wslot, dslot, rows, *sems):
    cid = lax.axis_index("c")
    sid = lax.axis_index("s")
    sms = sems[0:4]
    sws = sems[4:8]
    sds = sems[8:12]
    sgs = sems[12:16]
    scs = sems[16:20]
    emb_h = emb_hbm.at[cid]

    @pl.when(sid < NS - 1)
    def _():
        pltpu.sync_copy(zeros_hbm.at[pl.ds(0, ROWS_A)],
                        acc.at[pl.ds(sid * ROWS_A, ROWS_A)])

    @pl.when(sid == NS - 1)
    def _():
        pltpu.sync_copy(zeros_hbm.at[pl.ds(0, ROWS_B)],
                        acc.at[pl.ds(sid * ROWS_A, ROWS_B)])

    plsc.subcore_barrier()

    c00 = sid * NCHUNK

    def cidx(k):
        return c00 + jnp.minimum(k, NCHUNK - 1)

    def src_issue(k, u):
        pltpu.async_copy(src_hbm.at[cidx(k)], sslot.at[u], sms[u])
        pltpu.async_copy(w_hbm.at[cidx(k)], wslot.at[u], sws[u])

    def src_wait(k, u):
        pltpu.make_async_copy(src_hbm.at[cidx(k)], sslot.at[u],
                              sms[u]).wait()
        pltpu.make_async_copy(w_hbm.at[cidx(k)], wslot.at[u],
                              sws[u]).wait()

    def dst_issue(k, u):
        pltpu.async_copy(dst_hbm.at[cidx(k)], dslot.at[u], sds[u])

    def dst_wait(k, u):
        pltpu.make_async_copy(dst_hbm.at[cidx(k)], dslot.at[u],
                              sds[u]).wait()

    def gather_issue(k, u):
        pltpu.async_copy(emb_h.at[sslot.at[u]], rows.at[u], sgs[u])

    def gather_wait(k, u):
        pltpu.make_async_copy(emb_h.at[sslot.at[u]], rows.at[u],
                              sgs[u]).wait()

    def compute(u):
        for g in range(K // L):
            sl = pl.ds(g * L, L)
            wv = wslot[u, sl]
            for j in range(L):
                r = g * L + j
                sw = _lane_splat(wv, j)
                for c in range(DH // L):
                    cs = pl.ds(c * L, L)
                    rows[u, r, cs] = rows[u, r, cs] * sw

    def scatter_issue(u):
        pltpu.async_copy(rows.at[u], acc.at[dslot.at[u]], scs[u], add=True)

    def scatter_wait(u):
        pltpu.make_async_copy(rows.at[u], acc.at[dslot.at[u]],
                              scs[u]).wait()

    pltpu.sync_copy(src_hbm.at[cidx(0)], sslot.at[0])
    pltpu.sync_copy(w_hbm.at[cidx(0)], wslot.at[0])
    pltpu.sync_copy(src_hbm.at[cidx(1)], sslot.at[1])
    pltpu.sync_copy(w_hbm.at[cidx(1)], wslot.at[1])
    src_issue(2, 2)
    src_issue(3, 3)
    dst_issue(0, 0)
    dst_issue(1, 1)
    gather_issue(0, 0)
    gather_issue(1, 1)

    def quad_body(jj, carry):
        k0 = jj * 4
        for u in range(4):
            k = k0 + u
            u2 = (u + 2) % 4
            src_wait(k + 2, u2)

            @pl.when(k >= 2)
            def _():
                scatter_wait(u2)

            dst_issue(k + 2, u2)
            gather_issue(k + 2, u2)
            gather_wait(k, u)
            compute(u)
            dst_wait(k, u)
            scatter_issue(u)
            src_issue(k + 4, u)
        return carry

    lax.fori_loop(0, (NCHUNK - 1) // 4, quad_body, 0)

    gather_wait(NCHUNK - 1, 0)
    compute(0)
    dst_wait(NCHUNK - 1, 0)
    scatter_issue(0)

    src_wait(NCHUNK + 1, 2)
    src_wait(NCHUNK + 2, 3)
    gather_wait(NCHUNK, 1)
    dst_wait(NCHUNK, 1)
    scatter_wait(2)
    scatter_wait(3)
    scatter_wait(0)
    plsc.subcore_barrier()

    outc = out_hbm.at[cid]

    @pl.when(sid < NS - 1)
    def _():
        pltpu.sync_copy(acc.at[pl.ds(sid * ROWS_A, ROWS_A)],
                        outc.at[pl.ds(sid * ROWS_A, ROWS_A)])

    @pl.when(sid == NS - 1)
    def _():
        pltpu.sync_copy(acc.at[pl.ds(sid * ROWS_A, ROWS_B)],
                        outc.at[pl.ds(sid * ROWS_A, ROWS_B)])


_prop = pl.kernel(
    _prop_body,
    out_type=jax.ShapeDtypeStruct((NC, N, DH), jnp.float32),
    mesh=_MESH,
    scratch_types=[
        pltpu.VMEM_SHARED((N, DH), jnp.float32),
        pltpu.VMEM((4, K), jnp.int32),
        pltpu.VMEM((4, K), jnp.float32),
        pltpu.VMEM((4, K), jnp.int32),
        pltpu.VMEM((4, K, DH), jnp.float32),
    ] + [pltpu.SemaphoreType.DMA] * 20,
    compiler_params=pltpu.CompilerParams(use_tc_tiling_on_sc=False),
)

UPW = BATCH // (NC * NS)


def _users_body(e0_hbm, e1_hbm, e2_hbm, e3_hbm, users_hbm, out_hbm,
                idxbuf, b0, b1, b2, b3, ob, sem):
    wid = lax.axis_index("s") * NC + lax.axis_index("c")
    base = wid * UPW
    pltpu.sync_copy(users_hbm.at[pl.ds(base, UPW)], idxbuf)
    for h in range(NC):
        for tbl, buf in ((e0_hbm, b0), (e1_hbm, b1),
                         (e2_hbm, b2), (e3_hbm, b3)):
            pltpu.async_copy(tbl.at[h].at[idxbuf], buf, sem).wait()

        def row_body(r, carry, h=h):
            for c in range(DH // L):
                cs = pl.ds(c * L, L)
                os = pl.ds(h * DH + c * L, L)
                ob[r, os] = (b0[r, cs] + b1[r, cs]
                             + b2[r, cs] + b3[r, cs]) * 0.25
            return carry

        lax.fori_loop(0, UPW, row_body, 0, unroll=4)
    pltpu.sync_copy(ob, out_hbm.at[pl.ds(base, UPW)])


_users_mean = pl.kernel(
    _users_body,
    out_type=jax.ShapeDtypeStruct((BATCH, D), jnp.float32),
    mesh=_MESH,
    scratch_types=[
        pltpu.VMEM((UPW,), jnp.int32),
        pltpu.VMEM((UPW, DH), jnp.float32),
        pltpu.VMEM((UPW, DH), jnp.float32),
        pltpu.VMEM((UPW, DH), jnp.float32),
        pltpu.VMEM((UPW, DH), jnp.float32),
        pltpu.VMEM((UPW, D), jnp.float32),
        pltpu.SemaphoreType.DMA,
    ],
    compiler_params=pltpu.CompilerParams(use_tc_tiling_on_sc=False),
)

IB = 1000
UB = 128


def _items_mean_body(a0, a1, b0, b1, c0, c1, d0, d1, out_ref):
    left = (a0[...] + b0[...] + c0[...] + d0[...]) * 0.25
    right = (a1[...] + b1[...] + c1[...] + d1[...]) * 0.25
    out_ref[...] = jnp.concatenate([left, right], axis=1)


def _items_mean(e0, e1, e2, e3):
    nblk = NUM_ITEMS // IB
    off = NUM_USERS // IB
    in_specs = []
    for _ in range(4):
        in_specs.extend(
            pl.BlockSpec((None, IB, DH), lambda i, h=h: (h, off + i, 0))
            for h in range(NC))
    return pl.pallas_call(
        _items_mean_body,
        grid=(nblk,),
        in_specs=in_specs,
        out_specs=pl.BlockSpec((IB, D), lambda i: (i, 0)),
        out_shape=jax.ShapeDtypeStruct((NUM_ITEMS, D), jnp.float32),
    )(e0, e0, e1, e1, e2, e2, e3, e3)


def _rating_body(um_ref, items_ref, out_ref):
    out_ref[...] = lax.dot_general(
        um_ref[...], items_ref[...], (((1,), (1,)), ((), ())),
        preferred_element_type=jnp.float32)


def _rating(users_mean, items_mean):
    return pl.pallas_call(
        _rating_body,
        grid=(BATCH // UB,),
        in_specs=[
            pl.BlockSpec((UB, D), lambda i: (i, 0)),
            pl.BlockSpec((NUM_ITEMS, D), lambda i: (0, 0)),
        ],
        out_specs=pl.BlockSpec((UB, NUM_ITEMS), lambda i: (i, 0)),
        out_shape=jax.ShapeDtypeStruct((BATCH, NUM_ITEMS), jnp.float32),
    )(users_mean, items_mean)


@jax.jit
def kernel(user_emb, item_emb, edge_weight, edge_index, users):
    e0 = jnp.concatenate([user_emb, item_emb], axis=0)
    e0s = jnp.stack([e0[:, :DH], e0[:, DH:]], axis=0)
    tchunk = E // K
    srcs = edge_index[1].reshape(tchunk, K)
    dsts = edge_index[0].reshape(tchunk, K)
    wchunk = edge_weight.reshape(tchunk, K)
    zeros = jnp.zeros((ROWS_A, DH), jnp.float32)
    e1s = _prop(e0s, srcs, dsts, wchunk, zeros)
    e2s = _prop(e1s, srcs, dsts, wchunk, zeros)
    e3s = _prop(e2s, srcs, dsts, wchunk, zeros)
    users_mean = _users_mean(e0s, e1s, e2s, e3s, users)
    items_mean = _items_mean(e0s, e1s, e2s, e3s)
    return _rating(users_mean, items_mean)
